# baseline (speedup 1.0000x reference)
import functools

import jax
import jax.numpy as jnp
from jax import lax
from jax.experimental import pallas as pl
from jax.experimental.pallas import tpu as pltpu
from jax.experimental.pallas import tpu_sc as plsc

B = 4096
L = 200
D = 32
LANES = 16
NC = 2
NS = 16
NW = NC * NS

BBLK = B // NW
CL = 8
NCHUNK = L // CL
DBLK = D // 8
BGRP = BBLK // LANES


def _body(seqT_hbm, tok_hbm, pos_hbm, out_hbm, idx_v, gath_v, y_v, x_v, pos_v,
          gsem):
    wid = lax.axis_index("s") * NC + lax.axis_index("c")
    pltpu.sync_copy(pos_hbm, pos_v)
    b0 = wid * BBLK
    lane = lax.iota(jnp.int32, LANES)
    lane32 = lane * D

    def chunk_body(c, carry):
        l0 = c * CL
        pltpu.sync_copy(seqT_hbm.at[pl.ds(l0, CL), pl.ds(b0, BBLK)], idx_v)

        @plsc.parallel_loop(0, CL * BGRP, 1, unroll=4)
        def _perm_body(u):
            r = u // BGRP
            bg = u % BGRP
            t = idx_v[r, pl.ds(bg * LANES, LANES)]
            idx_v[r, pl.ds(bg * LANES, LANES)] = (
                (t & jnp.int32(-16384))
                | ((t & jnp.int32(4095)) << 2)
                | ((t & jnp.int32(16383)) >> 12)
            )

        copies = [
            pltpu.async_copy(
                tok_hbm.at[idx_v.at[r]],
                gath_v.at[pl.ds(r * BBLK, BBLK)],
                gsem,
            )
            for r in range(CL)
        ]
        for cp in copies:
            cp.wait()

        @plsc.parallel_loop(0, CL * BBLK, 1, unroll=8)
        def _skew_body(u):
            r = u // BBLK
            b = u % BBLK
            l = l0 + r
            base = u * D
            v0 = gath_v[u, pl.ds(0, LANES)] + pos_v[l, pl.ds(0, LANES)]
            v1 = gath_v[u, pl.ds(LANES, LANES)] + pos_v[l, pl.ds(LANES, LANES)]
            plsc.store_scatter(y_v, [base + ((lane + b) & 31)], v0)
            plsc.store_scatter(y_v, [base + ((lane + (b + LANES)) & 31)], v1)

        @plsc.parallel_loop(0, CL * D, 1, unroll=8)
        def _rd_body(t):
            r = t // D
            d = t % D
            for bg in range(BGRP):
                bvec = lane + bg * LANES
                u32 = (r * BBLK + bg * LANES) * D
                idx = (lane32 + u32) + ((bvec + d) & 31)
                v = plsc.load_gather(y_v, [idx])
                x_v[d // 8, r, d % 8, pl.ds(bg * LANES, LANES)] = v
        for db in range(DBLK):
            pltpu.sync_copy(x_v.at[db], out_hbm.at[pl.ds(l0, CL), db, wid])
        return carry

    lax.fori_loop(0, NCHUNK, chunk_body, 0)


VOCAB = 1000000
TC_CB = 16384
TC_GRID = (VOCAB + TC_CB - 1) // TC_CB
TC_ROWS = TC_GRID * (TC_CB // 4)
PERM_V = TC_GRID * TC_CB


def _tc_retile_body(in_ref, out_ref):
    x = in_ref[...]
    eye = jnp.eye(D, dtype=jnp.float32)
    y = lax.dot_general(x, eye, (((0,), (0,)), ((), ())),
                        preferred_element_type=jnp.float32)
    q = TC_CB // 4
    out_ref[...] = jnp.concatenate(
        [y[g * q:(g + 1) * q, :] for g in range(4)], axis=1)


def _tc_retile(tableT):
    return pl.pallas_call(
        _tc_retile_body,
        grid=(TC_GRID,),
        in_specs=[pl.BlockSpec((D, TC_CB), lambda i: (0, i))],
        out_specs=pl.BlockSpec((TC_CB // 4, 128), lambda i: (i, 0)),
        out_shape=jax.ShapeDtypeStruct((TC_ROWS, 128), jnp.float32),
    )(tableT)


def kernel(seq, token_table, pos_table):
    tok_lin = _tc_retile(jnp.transpose(token_table)).reshape(PERM_V, D)
    mesh = plsc.VectorSubcoreMesh(core_axis_name="c", subcore_axis_name="s")
    call = pl.kernel(
        _body,
        out_type=jax.ShapeDtypeStruct((L, DBLK, NW, 8, BBLK), jnp.float32),
        mesh=mesh,
        compiler_params=pltpu.CompilerParams(
            use_tc_tiling_on_sc=False, needs_layout_passes=False),
        scratch_types=[
            pltpu.VMEM((CL, BBLK), jnp.int32),
            pltpu.VMEM((CL * BBLK, D), jnp.float32),
            pltpu.VMEM((CL * BBLK * D,), jnp.float32),
            pltpu.VMEM((DBLK, CL, 8, BBLK), jnp.float32),
            pltpu.VMEM((L, D), jnp.float32),
            pltpu.SemaphoreType.DMA,
        ],
    )
    out5d = call(jnp.transpose(seq), tok_lin, pos_table)
    return out5d.transpose(2, 4, 0, 1, 3).reshape(B, L, D)


# speedup vs baseline: 1.0031x; 1.0031x over previous
import functools

import jax
import jax.numpy as jnp
from jax import lax
from jax.experimental import pallas as pl
from jax.experimental.pallas import tpu as pltpu
from jax.experimental.pallas import tpu_sc as plsc

B = 4096
L = 200
D = 32
LANES = 16
NC = 2
NS = 16
NW = NC * NS

BBLK = B // NW
CL = 8
NCHUNK = L // CL
DBLK = D // 8
BGRP = BBLK // LANES


def _body(seqT_hbm, tok_hbm, pos_hbm, out_hbm, idx_v, gath_v, y_v, x_v, pos_v,
          gsem):
    wid = lax.axis_index("s") * NC + lax.axis_index("c")
    pltpu.sync_copy(pos_hbm, pos_v)
    b0 = wid * BBLK
    lane = lax.iota(jnp.int32, LANES)
    lane32 = lane * D

    def chunk_body(c, carry):
        l0 = c * CL
        pltpu.sync_copy(seqT_hbm.at[pl.ds(l0, CL), pl.ds(b0, BBLK)], idx_v)

        @plsc.parallel_loop(0, CL * BGRP, 1, unroll=4)
        def _perm_body(u):
            r = u // BGRP
            bg = u % BGRP
            t = idx_v[r, pl.ds(bg * LANES, LANES)]
            idx_v[r, pl.ds(bg * LANES, LANES)] = (
                (t & jnp.int32(-16384))
                | ((t & jnp.int32(4095)) << 2)
                | ((t & jnp.int32(16383)) >> 12)
            )

        copies = [
            pltpu.async_copy(
                tok_hbm.at[idx_v.at[r]],
                gath_v.at[pl.ds(r * BBLK, BBLK)],
                gsem,
            )
            for r in range(CL)
        ]
        for cp in copies:
            cp.wait()

        @plsc.parallel_loop(0, CL * BBLK, 1, unroll=8)
        def _skew_body(u):
            r = u // BBLK
            b = u % BBLK
            l = l0 + r
            base = u * D
            v0 = gath_v[u, pl.ds(0, LANES)] + pos_v[l, pl.ds(0, LANES)]
            v1 = gath_v[u, pl.ds(LANES, LANES)] + pos_v[l, pl.ds(LANES, LANES)]
            plsc.store_scatter(y_v, [base + ((lane + b) & 31)], v0)
            plsc.store_scatter(y_v, [base + ((lane + (b + LANES)) & 31)], v1)

        @plsc.parallel_loop(0, CL * D, 1, unroll=8)
        def _rd_body(t):
            r = t // D
            d = t % D
            for bg in range(BGRP):
                bvec = lane + bg * LANES
                u32 = (r * BBLK + bg * LANES) * D
                idx = (lane32 + u32) + ((bvec + d) & 31)
                v = plsc.load_gather(y_v, [idx])
                x_v[d // 8, r, d % 8, pl.ds(bg * LANES, LANES)] = v
        for db in range(DBLK):
            pltpu.sync_copy(x_v.at[db], out_hbm.at[pl.ds(l0, CL), db, wid])
        return carry

    lax.fori_loop(0, NCHUNK, chunk_body, 0)


VOCAB = 1000000
TC_CB = 16384
TC_GRID = (VOCAB + TC_CB - 1) // TC_CB
TC_ROWS = TC_GRID * (TC_CB // 4)
PERM_V = TC_GRID * TC_CB


def _tc_retile_body(in_ref, out_ref):
    x = in_ref[...]
    y = x.T
    q = TC_CB // 4
    out_ref[...] = jnp.concatenate(
        [y[g * q:(g + 1) * q, :] for g in range(4)], axis=1)


def _tc_retile(tableT):
    return pl.pallas_call(
        _tc_retile_body,
        grid=(TC_GRID,),
        in_specs=[pl.BlockSpec((D, TC_CB), lambda i: (0, i))],
        out_specs=pl.BlockSpec((TC_CB // 4, 128), lambda i: (i, 0)),
        out_shape=jax.ShapeDtypeStruct((TC_ROWS, 128), jnp.float32),
    )(tableT)


def kernel(seq, token_table, pos_table):
    tok_lin = _tc_retile(jnp.transpose(token_table)).reshape(PERM_V, D)
    mesh = plsc.VectorSubcoreMesh(core_axis_name="c", subcore_axis_name="s")
    call = pl.kernel(
        _body,
        out_type=jax.ShapeDtypeStruct((L, DBLK, NW, 8, BBLK), jnp.float32),
        mesh=mesh,
        compiler_params=pltpu.CompilerParams(
            use_tc_tiling_on_sc=False, needs_layout_passes=False),
        scratch_types=[
            pltpu.VMEM((CL, BBLK), jnp.int32),
            pltpu.VMEM((CL * BBLK, D), jnp.float32),
            pltpu.VMEM((CL * BBLK * D,), jnp.float32),
            pltpu.VMEM((DBLK, CL, 8, BBLK), jnp.float32),
            pltpu.VMEM((L, D), jnp.float32),
            pltpu.SemaphoreType.DMA,
        ],
    )
    out5d = call(jnp.transpose(seq), tok_lin, pos_table)
    return out5d.transpose(2, 4, 0, 1, 3).reshape(B, L, D)
